# baseline (speedup 1.0000x reference)
import jax
import jax.numpy as jnp
from jax.experimental import pallas as pl
import jax.experimental.pallas.tpu as pltpu


def _fold128(t):
    acc = t[:, 0:128]
    for q in range(1, t.shape[1] // 128):
        acc = acc + t[:, q * 128:(q + 1) * 128]
    return acc


def _fold8(t):
    acc = t[0:8, :]
    for q in range(1, t.shape[0] // 8):
        acc = acc + t[q * 8:(q + 1) * 8, :]
    return acc


def _rowsum(t):
    return jnp.sum(_fold128(t), axis=1, keepdims=True)


def _make_kernel(pb):
    def _sinkhorn_kernel(x_ref, out_ref, idx_ref,
                         r_ref, c_ref, acc_ref):
        p = pl.program_id(0)
        i = pl.program_id(1)
        npass = pl.num_programs(0)
        ni = pl.num_programs(1)
        rows = pl.ds(i * pb, pb)
        s = jnp.exp(x_ref[...] * 50.0)

        @pl.when(p < npass - 1)
        def _():
            @pl.when(p == 0)
            def _():
                r_ref[rows] = 1.0 / _rowsum(s)

            @pl.when(p != 0)
            def _():
                r_ref[rows] = 1.0 / _rowsum(s * c_ref[...])

            part8 = _fold8(s * r_ref[rows])

            @pl.when(i == 0)
            def _():
                acc_ref[...] = part8

            @pl.when(i != 0)
            def _():
                acc_ref[...] += part8

            @pl.when(i == ni - 1)
            def _():
                c_ref[...] = 1.0 / jnp.sum(acc_ref[...], axis=0,
                                           keepdims=True)

        @pl.when(p == npass - 1)
        def _():
            o = s * r_ref[rows] * c_ref[...]
            out_ref[...] = o
            idx_ref[...] = jnp.argmax(o, axis=1).reshape(pb, 1).astype(
                jnp.int32)

    return _sinkhorn_kernel


def kernel(sims, batch_size=256):
    del batch_size
    num_row, num_col = sims.shape
    work = sims.T if num_row >= num_col else sims
    m, n = work.shape

    pb = min(256, m)
    npass = 6

    out, idx = pl.pallas_call(
        _make_kernel(pb),
        grid=(npass, m // pb),
        in_specs=[pl.BlockSpec((pb, n), lambda p, i: (i, 0))],
        out_specs=[
            pl.BlockSpec((pb, n),
                         lambda p, i, P=npass - 1:
                             (jnp.where(p == P, i, 0), 0)),
            pl.BlockSpec((pb, 1),
                         lambda p, i, P=npass - 1:
                             (jnp.where(p == P, i, 0), 0)),
        ],
        out_shape=[
            jax.ShapeDtypeStruct((m, n), jnp.float32),
            jax.ShapeDtypeStruct((m, 1), jnp.int32),
        ],
        scratch_shapes=[
            pltpu.VMEM((m, 1), jnp.float32),
            pltpu.VMEM((1, n), jnp.float32),
            pltpu.VMEM((8, n), jnp.float32),
        ],
    )(work)

    row_ids = jnp.arange(m, dtype=jnp.int32)
    col_ids = idx.reshape(m)
    if num_row >= num_col:
        indices = jnp.stack((col_ids, row_ids), axis=0)
    else:
        indices = jnp.stack((row_ids, col_ids), axis=0)
    values = jnp.ones((m,), dtype=jnp.float32)
    return (out, indices, values)


# speedup vs baseline: 1.0093x; 1.0093x over previous
import jax
import jax.numpy as jnp
from jax.experimental import pallas as pl
import jax.experimental.pallas.tpu as pltpu


def _fold128(t):
    acc = t[:, 0:128]
    for q in range(1, t.shape[1] // 128):
        acc = acc + t[:, q * 128:(q + 1) * 128]
    return acc


def _fold8(t):
    acc = t[0:8, :]
    for q in range(1, t.shape[0] // 8):
        acc = acc + t[q * 8:(q + 1) * 8, :]
    return acc


def _rowsum(t):
    return jnp.sum(_fold128(t), axis=1, keepdims=True)


def _sweep_first_kernel(x_ref, r_ref, c_ref, acc_ref):
    i = pl.program_id(0)
    ni = pl.num_programs(0)
    s = jnp.exp(x_ref[...] * 50.0)
    rp = 1.0 / _rowsum(s)
    r_ref[...] = rp
    part8 = _fold8(s * rp)

    @pl.when(i == 0)
    def _():
        acc_ref[...] = part8

    @pl.when(i != 0)
    def _():
        acc_ref[...] += part8

    @pl.when(i == ni - 1)
    def _():
        c_ref[...] = 1.0 / jnp.sum(acc_ref[...], axis=0, keepdims=True)


def _sweep_kernel(x_ref, cin_ref, r_ref, c_ref, acc_ref):
    i = pl.program_id(0)
    ni = pl.num_programs(0)
    s = jnp.exp(x_ref[...] * 50.0)
    rp = 1.0 / _rowsum(s * cin_ref[...])
    r_ref[...] = rp
    part8 = _fold8(s * rp)

    @pl.when(i == 0)
    def _():
        acc_ref[...] = part8

    @pl.when(i != 0)
    def _():
        acc_ref[...] += part8

    @pl.when(i == ni - 1)
    def _():
        c_ref[...] = 1.0 / jnp.sum(acc_ref[...], axis=0, keepdims=True)


def _output_kernel(x_ref, r_ref, c_ref, out_ref, idx_ref):
    o = jnp.exp(x_ref[...] * 50.0) * r_ref[...] * c_ref[...]
    out_ref[...] = o
    rows, n = o.shape
    acc = o[:, 0:128]
    for q in range(1, n // 128):
        acc = jnp.maximum(acc, o[:, q * 128:(q + 1) * 128])
    bm = jnp.max(acc, axis=1, keepdims=True)
    iota = jax.lax.broadcasted_iota(jnp.int32, (rows, n), 1)
    idx_ref[...] = jnp.min(jnp.where(o == bm, iota, n), axis=1,
                           keepdims=True)


def kernel(sims, batch_size=256):
    del batch_size
    num_row, num_col = sims.shape
    work = sims.T if num_row >= num_col else sims
    m, n = work.shape

    pb = min(512, m)
    po = min(256, m)

    grid = (m // pb,)
    x_spec = pl.BlockSpec((pb, n), lambda i: (i, 0))
    r_spec = pl.BlockSpec((pb, 1), lambda i: (i, 0))
    c_spec = pl.BlockSpec((1, n), lambda i: (0, 0))
    vec_shapes = [
        jax.ShapeDtypeStruct((m, 1), jnp.float32),
        jax.ShapeDtypeStruct((1, n), jnp.float32),
    ]
    acc = [pltpu.VMEM((8, n), jnp.float32)]

    r, c = pl.pallas_call(
        _sweep_first_kernel,
        grid=grid,
        in_specs=[x_spec],
        out_specs=[r_spec, c_spec],
        out_shape=vec_shapes,
        scratch_shapes=acc,
    )(work)

    sweep = pl.pallas_call(
        _sweep_kernel,
        grid=grid,
        in_specs=[x_spec, c_spec],
        out_specs=[r_spec, c_spec],
        out_shape=vec_shapes,
        scratch_shapes=acc,
    )
    for _ in range(4):
        r, c = sweep(work, c)

    out, idx = pl.pallas_call(
        _output_kernel,
        grid=(m // po,),
        in_specs=[
            pl.BlockSpec((po, n), lambda i: (i, 0)),
            pl.BlockSpec((po, 1), lambda i: (i, 0)),
            pl.BlockSpec((1, n), lambda i: (0, 0)),
        ],
        out_specs=[
            pl.BlockSpec((po, n), lambda i: (i, 0)),
            pl.BlockSpec((po, 1), lambda i: (i, 0)),
        ],
        out_shape=[
            jax.ShapeDtypeStruct((m, n), jnp.float32),
            jax.ShapeDtypeStruct((m, 1), jnp.int32),
        ],
    )(work, r, c)

    row_ids = jnp.arange(m, dtype=jnp.int32)
    col_ids = idx.reshape(m)
    if num_row >= num_col:
        indices = jnp.stack((col_ids, row_ids), axis=0)
    else:
        indices = jnp.stack((row_ids, col_ids), axis=0)
    values = jnp.ones((m,), dtype=jnp.float32)
    return (out, indices, values)
